# baseline (speedup 1.0000x reference)
import functools

import jax
import jax.numpy as jnp
from jax import lax
from jax.experimental import pallas as pl
from jax.experimental.pallas import tpu as pltpu
from jax.experimental.pallas import tpu_sc as plsc

N = 2048
H = 1024
I = 4096
E = 8
K = 2
B = 1024
T = (N * K + E * (B - 1)) // B + 1
P = T * B

NW = 32
AS = N * K


def _router_body(x_ref, gate_ref, dest_ref, w_ref, te_ref):
    x = x_ref[...]
    gate = gate_ref[...]
    logits = jax.lax.dot_general(
        x, gate, (((1,), (1,)), ((), ())), preferred_element_type=jnp.float32)
    m = jnp.max(logits, axis=-1, keepdims=True)
    p = jnp.exp(logits - m)
    probs = p / jnp.sum(p, axis=-1, keepdims=True)

    col = lax.broadcasted_iota(jnp.int32, (N, E), 1).astype(jnp.float32)
    big = jnp.float32(1e9)
    m1 = jnp.max(probs, axis=-1, keepdims=True)
    i1 = jnp.min(jnp.where(probs >= m1, col, big), axis=-1, keepdims=True)
    probs2 = jnp.where(col == i1, -1.0, probs)
    m2 = jnp.max(probs2, axis=-1, keepdims=True)
    i2 = jnp.min(jnp.where(probs2 >= m2, col, big), axis=-1, keepdims=True)

    denom = m1 + m2 + 1e-9
    w1 = m1 / denom
    w2 = m2 / denom

    M1 = (col == i1).astype(jnp.float32)
    M2 = (col == i2).astype(jnp.float32)
    S = M1 + M2
    c = S
    sh = 1
    while sh < N:
        c = c + jnp.concatenate(
            [jnp.zeros((sh, E), jnp.float32), c[:-sh, :]], axis=0)
        sh *= 2
    c_excl = c - S
    counts = jnp.sum(S, axis=0, keepdims=True)
    ptiles = jnp.floor((counts + (B - 1)) / B)
    r8 = lax.broadcasted_iota(jnp.int32, (E, E), 0)
    c8 = lax.broadcasted_iota(jnp.int32, (E, E), 1)
    upper = (r8 < c8).astype(jnp.float32)
    excl = jax.lax.dot_general(
        ptiles, upper, (((1,), (0,)), ((), ())),
        preferred_element_type=jnp.float32)
    poff = B * excl

    rank1 = jnp.sum(M1 * c_excl, axis=-1, keepdims=True)
    rank2 = jnp.sum(M2 * (c_excl + M1), axis=-1, keepdims=True)
    off1 = jnp.sum(M1 * poff, axis=-1, keepdims=True)
    off2 = jnp.sum(M2 * poff, axis=-1, keepdims=True)
    d1 = (off1 + rank1).astype(jnp.int32)
    d2 = (off2 + rank2).astype(jnp.int32)

    dest_ref[...] = jnp.concatenate([d1, d2], axis=-1)
    w_ref[...] = jnp.concatenate([w1, w2], axis=-1)

    ends = excl + ptiles
    ti = lax.broadcasted_iota(jnp.int32, (128, 1), 0).astype(jnp.float32)
    ge = (ti >= ends).astype(jnp.float32)
    te = jnp.minimum(jnp.sum(ge, axis=-1, keepdims=True), float(E - 1))
    col8 = lax.broadcasted_iota(jnp.int32, (1, E), 1)
    used = jnp.sum(ends * (col8 == (E - 1)), axis=-1, keepdims=True)
    te = jnp.where(ti == float(T), jnp.broadcast_to(used, (128, 1)), te)
    te_ref[...] = te.astype(jnp.int32)


def _run_router(hidden, gate_w, interpret=False):
    return pl.pallas_call(
        _router_body,
        out_shape=(
            jax.ShapeDtypeStruct((N, K), jnp.int32),
            jax.ShapeDtypeStruct((N, K), jnp.float32),
            jax.ShapeDtypeStruct((128, 1), jnp.int32),
        ),
        interpret=interpret,
    )(hidden, gate_w)


JH = I // 2


def _gemm_body(te_ref, x_ref, up_ref, down_ref, y_ref):
    i = pl.program_id(1)

    @pl.when(i < te_ref[T])
    def _():
        x = x_ref[...]
        h = jax.lax.dot_general(
            x, up_ref[0], (((1,), (1,)), ((), ())),
            preferred_element_type=jnp.float32)
        h = h * jax.nn.sigmoid(h)
        y_ref[0] = jax.lax.dot_general(
            h, down_ref[0], (((1,), (1,)), ((), ())),
            preferred_element_type=jnp.float32)


def _run_gemm(tile_expert, xg, up_w, down_w, interpret=False):
    grid_spec = pltpu.PrefetchScalarGridSpec(
        num_scalar_prefetch=1,
        grid=(2, T),
        in_specs=[
            pl.BlockSpec((B, H), lambda j, i, te: (i, 0)),
            pl.BlockSpec((1, JH, H), lambda j, i, te: (te[i], j, 0)),
            pl.BlockSpec((1, H, JH), lambda j, i, te: (te[i], 0, j)),
        ],
        out_specs=pl.BlockSpec((1, B, H), lambda j, i, te: (j, i, 0)),
    )
    return pl.pallas_call(
        _gemm_body,
        grid_spec=grid_spec,
        out_shape=jax.ShapeDtypeStruct((2, P, H), jnp.float32),
        compiler_params=pltpu.CompilerParams(
            dimension_semantics=("arbitrary", "arbitrary")),
        interpret=interpret,
    )(tile_expert, xg, up_w, down_w)


_CH = 64


def _dispatch_body(tok_hbm, dest_hbm, hidden_hbm, xg_hbm,
                   tok_v, dest_v, rows_v, sem):
    wid = lax.axis_index("s") * 2 + lax.axis_index("c")
    for r in range(AS // (NW * _CH)):
        base = wid * (AS // NW) + r * _CH
        pltpu.sync_copy(tok_hbm.at[pl.ds(base, _CH)], tok_v)
        pltpu.sync_copy(dest_hbm.at[pl.ds(base, _CH)], dest_v)
        pltpu.async_copy(hidden_hbm.at[tok_v], rows_v, sem).wait()
        pltpu.async_copy(rows_v, xg_hbm.at[dest_v], sem).wait()


def _run_dispatch(tok_ids, dest_flat, hidden):
    mesh = plsc.VectorSubcoreMesh(core_axis_name="c", subcore_axis_name="s")
    f = functools.partial(
        pl.kernel,
        out_type=jax.ShapeDtypeStruct((P, H), jnp.float32),
        mesh=mesh,
        scratch_types=[
            pltpu.VMEM((_CH,), jnp.int32),
            pltpu.VMEM((_CH,), jnp.int32),
            pltpu.VMEM((_CH, H), jnp.float32),
            pltpu.SemaphoreType.DMA,
        ],
    )(_dispatch_body)
    return f(tok_ids, dest_flat, hidden)


_CT = 16


_TPW = N // NW
_NR = _TPW // _CT


def _combine_body(d1_hbm, d2_hbm, w1_hbm, w2_hbm, y_hbm, out_hbm,
                  d1_v, d2_v, w1_v, w2_v,
                  buf1, buf1b, buf2, buf2b, sem):
    wid = lax.axis_index("s") * 2 + lax.axis_index("c")
    tb = wid * _TPW
    pltpu.sync_copy(d1_hbm.at[pl.ds(tb, _TPW)], d1_v.at[pl.ds(0, _TPW)])
    pltpu.sync_copy(d2_hbm.at[pl.ds(tb, _TPW)], d2_v.at[pl.ds(0, _TPW)])
    pltpu.sync_copy(w1_hbm.at[pl.ds(tb, _TPW)], w1_v.at[pl.ds(0, _TPW)])
    pltpu.sync_copy(w2_hbm.at[pl.ds(tb, _TPW)], w2_v.at[pl.ds(0, _TPW)])
    for c in range(_TPW // 16):
        sl = pl.ds(c * 16, 16)
        d1_v[pl.ds(_TPW + c * 16, 16)] = d1_v[sl] + P
        d2_v[pl.ds(_TPW + c * 16, 16)] = d2_v[sl] + P

    for r in range(_NR):
        cp1 = pltpu.async_copy(
            y_hbm.at[d1_v.at[pl.ds(r * _CT, _CT)]], buf1, sem)
        cp2 = pltpu.async_copy(
            y_hbm.at[d1_v.at[pl.ds(_TPW + r * _CT, _CT)]], buf1b, sem)
        cp3 = pltpu.async_copy(
            y_hbm.at[d2_v.at[pl.ds(r * _CT, _CT)]], buf2, sem)
        cp4 = pltpu.async_copy(
            y_hbm.at[d2_v.at[pl.ds(_TPW + r * _CT, _CT)]], buf2b, sem)
        cp1.wait()
        cp2.wait()
        cp3.wait()
        cp4.wait()

        def row_body(t, _):
            a = w1_v[pl.ds(r * _CT + t, 16)][0]
            b = w2_v[pl.ds(r * _CT + t, 16)][0]
            for c in range(H // 16):
                sl = pl.ds(c * 16, 16)
                buf1[t, sl] = (a * (buf1[t, sl] + buf1b[t, sl])
                               + b * (buf2[t, sl] + buf2b[t, sl]))
            return 0

        lax.fori_loop(0, _CT, row_body, 0)
        pltpu.sync_copy(buf1, out_hbm.at[pl.ds(tb + r * _CT, _CT)])


def _run_combine(d1, d2, w1, w2, y):
    mesh = plsc.VectorSubcoreMesh(core_axis_name="c", subcore_axis_name="s")
    f = functools.partial(
        pl.kernel,
        out_type=jax.ShapeDtypeStruct((N, H), jnp.float32),
        mesh=mesh,
        scratch_types=[
            pltpu.VMEM((2 * _TPW,), jnp.int32),
            pltpu.VMEM((2 * _TPW,), jnp.int32),
            pltpu.VMEM((_TPW + 16,), jnp.float32),
            pltpu.VMEM((_TPW + 16,), jnp.float32),
            pltpu.VMEM((_CT, H), jnp.float32),
            pltpu.VMEM((_CT, H), jnp.float32),
            pltpu.VMEM((_CT, H), jnp.float32),
            pltpu.VMEM((_CT, H), jnp.float32),
            pltpu.SemaphoreType.DMA,
        ],
    )(_combine_body)
    return f(d1, d2, w1, w2, y)


def kernel(hidden_states, gate_w, up_w, down_w):
    dest, w01, te = _run_router(hidden_states, gate_w)
    tile_expert = te[:T + 1, 0]
    tok_ids = jnp.arange(AS, dtype=jnp.int32) // K
    dest_flat = dest.reshape(AS)
    xg = _run_dispatch(tok_ids, dest_flat, hidden_states)
    y = _run_gemm(tile_expert, xg, up_w, down_w)
    out = _run_combine(dest[:, 0], dest[:, 1], w01[:, 0], w01[:, 1],
                       y.reshape(2 * P, H))
    return out


# speedup vs baseline: 1.1776x; 1.1776x over previous
import functools

import jax
import jax.numpy as jnp
from jax import lax
from jax.experimental import pallas as pl
from jax.experimental.pallas import tpu as pltpu
from jax.experimental.pallas import tpu_sc as plsc

N = 2048
H = 1024
I = 4096
E = 8
K = 2
B = 512
T = (N * K + E * (B - 1)) // B + 1
P = T * B

NW = 32
AS = N * K


def _router_body(x_ref, gate_ref, dest_ref, w_ref, te_ref):
    x = x_ref[...]
    gate = gate_ref[...]
    logits = jax.lax.dot_general(
        x, gate, (((1,), (1,)), ((), ())), preferred_element_type=jnp.float32)
    m = jnp.max(logits, axis=-1, keepdims=True)
    p = jnp.exp(logits - m)
    probs = p / jnp.sum(p, axis=-1, keepdims=True)

    col = lax.broadcasted_iota(jnp.int32, (N, E), 1).astype(jnp.float32)
    big = jnp.float32(1e9)
    m1 = jnp.max(probs, axis=-1, keepdims=True)
    i1 = jnp.min(jnp.where(probs >= m1, col, big), axis=-1, keepdims=True)
    probs2 = jnp.where(col == i1, -1.0, probs)
    m2 = jnp.max(probs2, axis=-1, keepdims=True)
    i2 = jnp.min(jnp.where(probs2 >= m2, col, big), axis=-1, keepdims=True)

    denom = m1 + m2 + 1e-9
    w1 = m1 / denom
    w2 = m2 / denom

    M1 = (col == i1).astype(jnp.float32)
    M2 = (col == i2).astype(jnp.float32)
    S = M1 + M2
    c = S
    sh = 1
    while sh < N:
        c = c + jnp.concatenate(
            [jnp.zeros((sh, E), jnp.float32), c[:-sh, :]], axis=0)
        sh *= 2
    c_excl = c - S
    counts = jnp.sum(S, axis=0, keepdims=True)
    ptiles = jnp.floor((counts + (B - 1)) / B)
    r8 = lax.broadcasted_iota(jnp.int32, (E, E), 0)
    c8 = lax.broadcasted_iota(jnp.int32, (E, E), 1)
    upper = (r8 < c8).astype(jnp.float32)
    excl = jax.lax.dot_general(
        ptiles, upper, (((1,), (0,)), ((), ())),
        preferred_element_type=jnp.float32)
    poff = B * excl

    rank1 = jnp.sum(M1 * c_excl, axis=-1, keepdims=True)
    rank2 = jnp.sum(M2 * (c_excl + M1), axis=-1, keepdims=True)
    off1 = jnp.sum(M1 * poff, axis=-1, keepdims=True)
    off2 = jnp.sum(M2 * poff, axis=-1, keepdims=True)
    d1 = (off1 + rank1).astype(jnp.int32)
    d2 = (off2 + rank2).astype(jnp.int32)

    dest_ref[...] = jnp.concatenate([d1, d2], axis=-1)
    w_ref[...] = jnp.concatenate([w1, w2], axis=-1)

    ends = excl + ptiles
    ti = lax.broadcasted_iota(jnp.int32, (128, 1), 0).astype(jnp.float32)
    ge = (ti >= ends).astype(jnp.float32)
    te = jnp.minimum(jnp.sum(ge, axis=-1, keepdims=True), float(E - 1))
    col8 = lax.broadcasted_iota(jnp.int32, (1, E), 1)
    used = jnp.sum(ends * (col8 == (E - 1)), axis=-1, keepdims=True)
    te = jnp.where(ti == float(T), jnp.broadcast_to(used, (128, 1)), te)
    te_ref[...] = te.astype(jnp.int32)


def _run_router(hidden, gate_w, interpret=False):
    return pl.pallas_call(
        _router_body,
        out_shape=(
            jax.ShapeDtypeStruct((N, K), jnp.int32),
            jax.ShapeDtypeStruct((N, K), jnp.float32),
            jax.ShapeDtypeStruct((128, 1), jnp.int32),
        ),
        interpret=interpret,
    )(hidden, gate_w)


JH = I // 2


def _gemm_body(te_ref, x_ref, up_ref, down_ref, y_ref):
    i = pl.program_id(1)

    @pl.when(i < te_ref[T])
    def _():
        x = x_ref[...]
        h = jax.lax.dot_general(
            x, up_ref[0], (((1,), (1,)), ((), ())),
            preferred_element_type=jnp.float32)
        h = h * jax.nn.sigmoid(h)
        y_ref[0] = jax.lax.dot_general(
            h, down_ref[0], (((1,), (1,)), ((), ())),
            preferred_element_type=jnp.float32)


def _run_gemm(tile_expert, xg, up_w, down_w, interpret=False):
    grid_spec = pltpu.PrefetchScalarGridSpec(
        num_scalar_prefetch=1,
        grid=(2, T),
        in_specs=[
            pl.BlockSpec((B, H), lambda j, i, te: (i, 0)),
            pl.BlockSpec((1, JH, H), lambda j, i, te: (te[i], j, 0)),
            pl.BlockSpec((1, H, JH), lambda j, i, te: (te[i], 0, j)),
        ],
        out_specs=pl.BlockSpec((1, B, H), lambda j, i, te: (j, i, 0)),
    )
    return pl.pallas_call(
        _gemm_body,
        grid_spec=grid_spec,
        out_shape=jax.ShapeDtypeStruct((2, P, H), jnp.float32),
        compiler_params=pltpu.CompilerParams(
            dimension_semantics=("arbitrary", "arbitrary")),
        interpret=interpret,
    )(tile_expert, xg, up_w, down_w)


_TPW = N // NW


def _dispatch_body(d1_hbm, d2_hbm, hidden_hbm, xg_hbm,
                   d1_v, d2_v, rows_v, sem):
    wid = lax.axis_index("s") * 2 + lax.axis_index("c")
    tb = wid * _TPW
    ra = pltpu.async_copy(hidden_hbm.at[pl.ds(tb, _TPW)], rows_v, sem)
    pltpu.sync_copy(d1_hbm.at[pl.ds(tb, _TPW)], d1_v)
    pltpu.sync_copy(d2_hbm.at[pl.ds(tb, _TPW)], d2_v)
    ra.wait()
    c1 = pltpu.async_copy(rows_v, xg_hbm.at[d1_v], sem)
    c2 = pltpu.async_copy(rows_v, xg_hbm.at[d2_v], sem)
    c1.wait()
    c2.wait()


def _run_dispatch(d1, d2, hidden):
    mesh = plsc.VectorSubcoreMesh(core_axis_name="c", subcore_axis_name="s")
    f = functools.partial(
        pl.kernel,
        out_type=jax.ShapeDtypeStruct((P, H), jnp.float32),
        mesh=mesh,
        scratch_types=[
            pltpu.VMEM((_TPW,), jnp.int32),
            pltpu.VMEM((_TPW,), jnp.int32),
            pltpu.VMEM((_TPW, H), jnp.float32),
            pltpu.SemaphoreType.DMA,
        ],
    )(_dispatch_body)
    return f(d1, d2, hidden)


_CT = 8
_NR = _TPW // _CT


def _combine_body(d1_hbm, d2_hbm, w1_hbm, w2_hbm, y_hbm, out_hbm,
                  d1_v, d2_v, w1_v, w2_v,
                  b1a0, b1b0, b2a0, b2b0, b1a1, b1b1, b2a1, b2b1,
                  gsem0, gsem1, ssem0, ssem1):
    wid = lax.axis_index("s") * 2 + lax.axis_index("c")
    tb = wid * _TPW
    pltpu.sync_copy(d1_hbm.at[pl.ds(tb, _TPW)], d1_v.at[pl.ds(0, _TPW)])
    pltpu.sync_copy(d2_hbm.at[pl.ds(tb, _TPW)], d2_v.at[pl.ds(0, _TPW)])
    pltpu.sync_copy(w1_hbm.at[pl.ds(tb, _TPW)], w1_v.at[pl.ds(0, _TPW)])
    pltpu.sync_copy(w2_hbm.at[pl.ds(tb, _TPW)], w2_v.at[pl.ds(0, _TPW)])
    for c in range(_TPW // 16):
        sl = pl.ds(c * 16, 16)
        d1_v[pl.ds(_TPW + c * 16, 16)] = d1_v[sl] + P
        d2_v[pl.ds(_TPW + c * 16, 16)] = d2_v[sl] + P

    sets = [(b1a0, b1b0, b2a0, b2b0, gsem0, ssem0),
            (b1a1, b1b1, b2a1, b2b1, gsem1, ssem1)]

    def issue(r):
        S = sets[r % 2]
        idxs = (d1_v.at[pl.ds(r * _CT, _CT)],
                d1_v.at[pl.ds(_TPW + r * _CT, _CT)],
                d2_v.at[pl.ds(r * _CT, _CT)],
                d2_v.at[pl.ds(_TPW + r * _CT, _CT)])
        return [pltpu.async_copy(y_hbm.at[ix], buf, S[4])
                for ix, buf in zip(idxs, S[:4])]

    descs = issue(0)
    stores = [None, None]
    for r in range(_NR):
        S = sets[r % 2]
        for cp in descs:
            cp.wait()
        if r + 1 < _NR:
            o = (r + 1) % 2
            if stores[o] is not None:
                stores[o].wait()
            descs = issue(r + 1)
        buf1, buf1b, buf2, buf2b = S[:4]

        def row_body(t, _):
            a = w1_v[pl.ds(r * _CT + t, 16)][0]
            b = w2_v[pl.ds(r * _CT + t, 16)][0]
            for c in range(H // 16):
                sl = pl.ds(c * 16, 16)
                buf1[t, sl] = (a * (buf1[t, sl] + buf1b[t, sl])
                               + b * (buf2[t, sl] + buf2b[t, sl]))
            return 0

        lax.fori_loop(0, _CT, row_body, 0)
        stores[r % 2] = pltpu.async_copy(
            buf1, out_hbm.at[pl.ds(tb + r * _CT, _CT)], S[5])
    stores[0].wait()
    stores[1].wait()


def _run_combine(d1, d2, w1, w2, y):
    mesh = plsc.VectorSubcoreMesh(core_axis_name="c", subcore_axis_name="s")
    f = functools.partial(
        pl.kernel,
        out_type=jax.ShapeDtypeStruct((N, H), jnp.float32),
        mesh=mesh,
        scratch_types=[
            pltpu.VMEM((2 * _TPW,), jnp.int32),
            pltpu.VMEM((2 * _TPW,), jnp.int32),
            pltpu.VMEM((_TPW + 16,), jnp.float32),
            pltpu.VMEM((_TPW + 16,), jnp.float32),
            pltpu.VMEM((_CT, H), jnp.float32),
            pltpu.VMEM((_CT, H), jnp.float32),
            pltpu.VMEM((_CT, H), jnp.float32),
            pltpu.VMEM((_CT, H), jnp.float32),
            pltpu.VMEM((_CT, H), jnp.float32),
            pltpu.VMEM((_CT, H), jnp.float32),
            pltpu.VMEM((_CT, H), jnp.float32),
            pltpu.VMEM((_CT, H), jnp.float32),
            pltpu.SemaphoreType.DMA,
            pltpu.SemaphoreType.DMA,
            pltpu.SemaphoreType.DMA,
            pltpu.SemaphoreType.DMA,
        ],
    )(_combine_body)
    return f(d1, d2, w1, w2, y)


def kernel(hidden_states, gate_w, up_w, down_w):
    dest, w01, te = _run_router(hidden_states, gate_w)
    tile_expert = te[:T + 1, 0]
    d1 = dest[:, 0]
    d2 = dest[:, 1]
    xg = _run_dispatch(d1, d2, hidden_states)
    y = _run_gemm(tile_expert, xg, up_w, down_w)
    out = _run_combine(d1, d2, w01[:, 0], w01[:, 1], y.reshape(2 * P, H))
    return out
